# baseline (speedup 1.0000x reference)
import functools

import jax
import jax.numpy as jnp
from jax import lax
from jax.experimental import pallas as pl
from jax.experimental.pallas import tpu as pltpu
from jax.experimental.pallas import tpu_sc as plsc

N = 50000
MUL = 128
NUM_COEFFS = 9
DIM = NUM_COEFFS * MUL
LANES = 16
NV = DIM // LANES
GRP = 24
NC = 2
NS = 16
NW = NC * NS
CH = 16
NCHUNK = N // CH
TRIPS = (NCHUNK + NW - 1) // NW


@jax.jit
def _pack(x, pack_index):
    @functools.partial(
        pl.kernel,
        mesh=plsc.VectorSubcoreMesh(core_axis_name="c", subcore_axis_name="s"),
        out_type=jax.ShapeDtypeStruct((NUM_COEFFS, N, MUL), jnp.float32),
        scratch_types=[
            pltpu.VMEM((DIM,), jnp.int32),
            pltpu.VMEM((CH * NUM_COEFFS, MUL), jnp.float32),
            pltpu.VMEM((CH * NUM_COEFFS, MUL), jnp.float32),
            pltpu.VMEM((NUM_COEFFS * CH, MUL), jnp.float32),
            pltpu.VMEM((NUM_COEFFS * CH, MUL), jnp.float32),
            pltpu.SemaphoreType.DMA,
            pltpu.SemaphoreType.DMA,
            pltpu.SemaphoreType.DMA,
            pltpu.SemaphoreType.DMA,
        ],
        compiler_params=pltpu.CompilerParams(
            use_tc_tiling_on_sc=True, needs_layout_passes=False
        ),
    )
    def k(x_hbm, idx_hbm, out_hbm, idx_v, xb0, xb1, ob0, ob1,
          isem0, isem1, osem0, osem1):
        xbufs = (xb0, xb1)
        obufs = (ob0, ob1)
        isems = (isem0, isem1)
        osems = (osem0, osem1)
        wid = lax.axis_index("s") * NC + lax.axis_index("c")
        pltpu.sync_copy(idx_hbm, idx_v)
        iota = lax.iota(jnp.int32, LANES)
        zv = jnp.zeros((LANES,), jnp.int32)
        nch_w = (NCHUNK - 1 - wid) // NW + 1

        def start_in(t, b):
            @pl.when(t < nch_w)
            def _():
                row0 = (wid + t * NW) * CH
                pltpu.async_copy(
                    x_hbm.at[pl.ds(row0, CH)],
                    xbufs[b].reshape(CH, DIM),
                    isems[b],
                )

        def wait_in(t, b):
            @pl.when(t < nch_w)
            def _():
                pltpu.make_async_copy(
                    x_hbm.at[pl.ds(0, CH)], xbufs[b].reshape(CH, DIM), isems[b]
                ).wait()

        def start_out(t, b):
            @pl.when(t < nch_w)
            def _():
                row0 = (wid + t * NW) * CH
                pltpu.async_copy(
                    obufs[b].reshape(NUM_COEFFS, CH, MUL),
                    out_hbm.at[:, pl.ds(row0, CH), :],
                    osems[b],
                )

        def wait_out(b, cond):
            @pl.when(cond)
            def _():
                pltpu.make_async_copy(
                    obufs[b].reshape(NUM_COEFFS, CH, MUL),
                    out_hbm.at[:, pl.ds(0, CH), :],
                    osems[b],
                ).wait()

        def compute(t, b):
            @pl.when(t < nch_w)
            def _():
                xbuf = xbufs[b]
                obuf = obufs[b]
                for g in range(NV // GRP):
                    idx_g = tuple(
                        plsc.load_gather(idx_v, [iota + (g * GRP + j) * LANES])
                        for j in range(GRP)
                    )

                    @plsc.parallel_loop(0, CH, unroll=1, carry=idx_g)
                    def row_body(r, idxs, g=g, xbuf=xbuf, obuf=obuf):
                        rbase = r * DIM
                        orbase = r * MUL + iota
                        vals = [
                            plsc.load_gather(xbuf, [zv, idxs[j] + rbase])
                            for j in range(GRP)
                        ]
                        for j in range(GRP):
                            v = g * GRP + j
                            oc = (v // 8) * CH * MUL + (v % 8) * LANES
                            plsc.store_scatter(
                                obuf, [zv, orbase + oc], vals[j]
                            )
                        return idxs

        start_in(0, 0)

        def super_body(i, carry):
            for b in range(2):
                t = i * 2 + b
                wait_in(t, b)
                start_in(t + 1, 1 - b)
                wait_out(b, (t >= 2) & (t < nch_w))
                compute(t, b)
                start_out(t, b)
            return carry

        assert TRIPS % 2 == 0
        lax.fori_loop(0, TRIPS // 2, super_body, 0, unroll=1)
        for b in range(2):
            tb = ((nch_w - 1 - b) // 2) * 2 + b
            wait_out(b, (tb >= 0) & (tb < nch_w) & (tb >= nch_w - 2))

    return k(x, pack_index)


def kernel(x_flat, pack_index):
    out = _pack(x_flat, pack_index.astype(jnp.int32))
    return out.transpose(1, 0, 2)


# speedup vs baseline: 1.3766x; 1.3766x over previous
import functools

import jax
import jax.numpy as jnp
from jax import lax
from jax.experimental import pallas as pl
from jax.experimental.pallas import tpu as pltpu
from jax.experimental.pallas import tpu_sc as plsc

N = 50000
MUL = 128
NUM_COEFFS = 9
DIM = NUM_COEFFS * MUL
LANES = 16
NV = DIM // LANES
GRP = 12
NC = 2
NS = 16
NW = NC * NS
CH = 16
NCHUNK = N // CH
TRIPS = (NCHUNK + NW - 1) // NW


@jax.jit
def _pack(x, pack_index):
    @functools.partial(
        pl.kernel,
        mesh=plsc.VectorSubcoreMesh(core_axis_name="c", subcore_axis_name="s"),
        out_type=jax.ShapeDtypeStruct((NUM_COEFFS, N, MUL), jnp.float32),
        scratch_types=[
            pltpu.VMEM((DIM,), jnp.int32),
            pltpu.VMEM((CH * NUM_COEFFS, MUL), jnp.float32),
            pltpu.VMEM((CH * NUM_COEFFS, MUL), jnp.float32),
            pltpu.VMEM((NUM_COEFFS * CH, MUL), jnp.float32),
            pltpu.VMEM((NUM_COEFFS * CH, MUL), jnp.float32),
            pltpu.SemaphoreType.DMA,
            pltpu.SemaphoreType.DMA,
            pltpu.SemaphoreType.DMA,
            pltpu.SemaphoreType.DMA,
        ],
        compiler_params=pltpu.CompilerParams(
            use_tc_tiling_on_sc=True, needs_layout_passes=False
        ),
    )
    def k(x_hbm, idx_hbm, out_hbm, idx_v, xb0, xb1, ob0, ob1,
          isem0, isem1, osem0, osem1):
        xbufs = (xb0, xb1)
        obufs = (ob0, ob1)
        isems = (isem0, isem1)
        osems = (osem0, osem1)
        wid = lax.axis_index("s") * NC + lax.axis_index("c")
        pltpu.sync_copy(idx_hbm, idx_v)
        iota = lax.iota(jnp.int32, LANES)
        zv = jnp.zeros((LANES,), jnp.int32)
        nch_w = (NCHUNK - 1 - wid) // NW + 1

        def start_in(t, b):
            @pl.when(t < nch_w)
            def _():
                row0 = (wid + t * NW) * CH
                pltpu.async_copy(
                    x_hbm.at[pl.ds(row0, CH)],
                    xbufs[b].reshape(CH, DIM),
                    isems[b],
                )

        def wait_in(t, b):
            @pl.when(t < nch_w)
            def _():
                pltpu.make_async_copy(
                    x_hbm.at[pl.ds(0, CH)], xbufs[b].reshape(CH, DIM), isems[b]
                ).wait()

        def start_out(t, b):
            @pl.when(t < nch_w)
            def _():
                row0 = (wid + t * NW) * CH
                pltpu.async_copy(
                    obufs[b].reshape(NUM_COEFFS, CH, MUL),
                    out_hbm.at[:, pl.ds(row0, CH), :],
                    osems[b],
                )

        def wait_out(b, cond):
            @pl.when(cond)
            def _():
                pltpu.make_async_copy(
                    obufs[b].reshape(NUM_COEFFS, CH, MUL),
                    out_hbm.at[:, pl.ds(0, CH), :],
                    osems[b],
                ).wait()

        def compute(t, b):
            @pl.when(t < nch_w)
            def _():
                xbuf = xbufs[b]
                obuf = obufs[b]
                for g in range(NV // GRP):
                    idx_g = tuple(
                        plsc.load_gather(idx_v, [iota + (g * GRP + j) * LANES])
                        for j in range(GRP)
                    )

                    @plsc.parallel_loop(0, CH, unroll=2, carry=idx_g)
                    def row_body(r, idxs, g=g, xbuf=xbuf, obuf=obuf):
                        rbase = r * DIM
                        orbase = r * MUL + iota
                        vals = [
                            plsc.load_gather(xbuf, [zv, idxs[j] + rbase])
                            for j in range(GRP)
                        ]
                        for j in range(GRP):
                            v = g * GRP + j
                            oc = (v // 8) * CH * MUL + (v % 8) * LANES
                            plsc.store_scatter(
                                obuf, [zv, orbase + oc], vals[j]
                            )
                        return idxs

        start_in(0, 0)

        def super_body(i, carry):
            for b in range(2):
                t = i * 2 + b
                wait_in(t, b)
                start_in(t + 1, 1 - b)
                wait_out(b, (t >= 2) & (t < nch_w))
                compute(t, b)
                start_out(t, b)
            return carry

        assert TRIPS % 2 == 0
        lax.fori_loop(0, TRIPS // 2, super_body, 0, unroll=1)
        for b in range(2):
            tb = ((nch_w - 1 - b) // 2) * 2 + b
            wait_out(b, (tb >= 0) & (tb < nch_w) & (tb >= nch_w - 2))

    return k(x, pack_index)


def kernel(x_flat, pack_index):
    out = _pack(x_flat, pack_index.astype(jnp.int32))
    return out.transpose(1, 0, 2)


# speedup vs baseline: 1.5521x; 1.1275x over previous
import functools

import jax
import jax.numpy as jnp
from jax import lax
from jax.experimental import pallas as pl
from jax.experimental.pallas import tpu as pltpu
from jax.experimental.pallas import tpu_sc as plsc

N = 50000
MUL = 128
NUM_COEFFS = 9
DIM = NUM_COEFFS * MUL
LANES = 16
NV = DIM // LANES
GRP = 12
NC = 2
NS = 16
NW = NC * NS
CH = 16
NCHUNK = N // CH
TRIPS = (NCHUNK + NW - 1) // NW


@jax.jit
def _pack(x, pack_index):
    @functools.partial(
        pl.kernel,
        mesh=plsc.VectorSubcoreMesh(core_axis_name="c", subcore_axis_name="s"),
        out_type=jax.ShapeDtypeStruct((NUM_COEFFS, N, MUL), jnp.float32),
        scratch_types=[
            pltpu.VMEM((DIM,), jnp.int32),
            pltpu.VMEM((CH * NUM_COEFFS, MUL), jnp.float32),
            pltpu.VMEM((CH * NUM_COEFFS, MUL), jnp.float32),
            pltpu.VMEM((NUM_COEFFS * CH, MUL), jnp.float32),
            pltpu.VMEM((NUM_COEFFS * CH, MUL), jnp.float32),
            pltpu.SemaphoreType.DMA,
            pltpu.SemaphoreType.DMA,
            pltpu.SemaphoreType.DMA,
            pltpu.SemaphoreType.DMA,
        ],
        compiler_params=pltpu.CompilerParams(
            use_tc_tiling_on_sc=True, needs_layout_passes=False
        ),
    )
    def k(x_hbm, idx_hbm, out_hbm, idx_v, xb0, xb1, ob0, ob1,
          isem0, isem1, osem0, osem1):
        xbufs = (xb0, xb1)
        obufs = (ob0, ob1)
        isems = (isem0, isem1)
        osems = (osem0, osem1)
        wid = lax.axis_index("s") * NC + lax.axis_index("c")
        pltpu.sync_copy(idx_hbm, idx_v)
        iota = lax.iota(jnp.int32, LANES)
        zv = jnp.zeros((LANES,), jnp.int32)
        nch_w = (NCHUNK - 1 - wid) // NW + 1

        def start_in(t, b):
            @pl.when(t < nch_w)
            def _():
                row0 = (wid + t * NW) * CH
                pltpu.async_copy(
                    x_hbm.at[pl.ds(row0, CH)],
                    xbufs[b].reshape(CH, DIM),
                    isems[b],
                )

        def wait_in(t, b):
            @pl.when(t < nch_w)
            def _():
                pltpu.make_async_copy(
                    x_hbm.at[pl.ds(0, CH)], xbufs[b].reshape(CH, DIM), isems[b]
                ).wait()

        def start_out(t, b):
            @pl.when(t < nch_w)
            def _():
                row0 = (wid + t * NW) * CH
                pltpu.async_copy(
                    obufs[b].reshape(NUM_COEFFS, CH, MUL),
                    out_hbm.at[:, pl.ds(row0, CH), :],
                    osems[b],
                )

        def wait_out(b, cond):
            @pl.when(cond)
            def _():
                pltpu.make_async_copy(
                    obufs[b].reshape(NUM_COEFFS, CH, MUL),
                    out_hbm.at[:, pl.ds(0, CH), :],
                    osems[b],
                ).wait()

        def compute(t, b):
            @pl.when(t < nch_w)
            def _():
                xbuf = xbufs[b]
                obuf = obufs[b]
                for g in range(NV // GRP):
                    idx_g = tuple(
                        plsc.load_gather(idx_v, [iota + (g * GRP + j) * LANES])
                        for j in range(GRP)
                    )

                    @plsc.parallel_loop(0, CH, unroll=1, carry=idx_g)
                    def row_body(r, idxs, g=g, xbuf=xbuf, obuf=obuf):
                        rbase = r * DIM
                        orbase = r * MUL + iota
                        vals = [
                            plsc.load_gather(xbuf, [zv, idxs[j] + rbase])
                            for j in range(GRP)
                        ]
                        for j in range(GRP):
                            v = g * GRP + j
                            oc = (v // 8) * CH * MUL + (v % 8) * LANES
                            plsc.store_scatter(
                                obuf, [zv, orbase + oc], vals[j]
                            )
                        return idxs

        start_in(0, 0)

        def super_body(i, carry):
            for b in range(2):
                t = i * 2 + b
                wait_in(t, b)
                start_in(t + 1, 1 - b)
                wait_out(b, (t >= 2) & (t < nch_w))
                compute(t, b)
                start_out(t, b)
            return carry

        assert TRIPS % 2 == 0
        lax.fori_loop(0, TRIPS // 2, super_body, 0, unroll=1)
        for b in range(2):
            tb = ((nch_w - 1 - b) // 2) * 2 + b
            wait_out(b, (tb >= 0) & (tb < nch_w) & (tb >= nch_w - 2))

    return k(x, pack_index)


def kernel(x_flat, pack_index):
    out = _pack(x_flat, pack_index.astype(jnp.int32))
    return out.transpose(1, 0, 2)
